# baseline (speedup 1.0000x reference)
import jax
import jax.numpy as jnp
from jax.experimental import pallas as pl
from jax.experimental.pallas import tpu as pltpu

_LANE = 128


def _round_up(n, m):
    return ((n + m - 1) // m) * m


def _mlp_kernel(x_ref, w1_ref, b1_ref, w2_ref, b2_ref, wr_ref, br_ref, o_ref):
    h = jnp.dot(x_ref[...], w1_ref[...], preferred_element_type=jnp.float32)
    h = jnp.maximum(h + b1_ref[...], 0.0).astype(jnp.bfloat16)
    h = jnp.dot(h, w2_ref[...], preferred_element_type=jnp.float32)
    h = jnp.maximum(h + b2_ref[...], 0.0).astype(jnp.bfloat16)
    y = jnp.dot(h, wr_ref[...], preferred_element_type=jnp.float32)
    o_ref[...] = (y + br_ref[...]).astype(o_ref.dtype)


def kernel(x, w1, b1, w2, b2, wr, br):
    B, z_dim = x.shape
    h0_dim, h1_dim, x_dim = w1.shape[1], w2.shape[1], wr.shape[1]

    z_p = _round_up(z_dim, _LANE)
    h0_p = _round_up(h0_dim, _LANE)
    h1_p = _round_up(h1_dim, _LANE)
    x_p = _round_up(x_dim, _LANE)

    tm = 512 if B >= 512 else _round_up(max(B, 1), 8)
    B_p = _round_up(B, tm)
    nb = B_p // tm

    def pad2(a, rows, cols):
        if a.shape == (rows, cols):
            return a
        return jnp.pad(a, ((0, rows - a.shape[0]), (0, cols - a.shape[1])))

    bf16 = jnp.bfloat16
    x_pad = pad2(x, B_p, z_p).astype(bf16)
    w1_p = pad2(w1, z_p, h0_p).astype(bf16)
    w2_p = pad2(w2, h0_p, h1_p).astype(bf16)
    wr_p = pad2(wr, h1_p, x_p).astype(bf16)
    b1_p = pad2(b1, 1, h0_p)
    b2_p = pad2(b2, 1, h1_p)
    br_p = pad2(br, 1, x_p)

    resident = lambda shape: pl.BlockSpec(shape, lambda i: (0, 0))

    out = pl.pallas_call(
        _mlp_kernel,
        out_shape=jax.ShapeDtypeStruct((B_p, x_p), x.dtype),
        grid=(nb,),
        in_specs=[
            pl.BlockSpec((tm, z_p), lambda i: (i, 0)),
            resident((z_p, h0_p)), resident((1, h0_p)),
            resident((h0_p, h1_p)), resident((1, h1_p)),
            resident((h1_p, x_p)), resident((1, x_p)),
        ],
        out_specs=pl.BlockSpec((tm, x_p), lambda i: (i, 0)),
        compiler_params=pltpu.CompilerParams(
            dimension_semantics=("parallel",),
            vmem_limit_bytes=64 * 1024 * 1024,
        ),
    )(x_pad, w1_p, b1_p, w2_p, b2_p, wr_p, br_p)

    return out[:B, :x_dim]


# speedup vs baseline: 1.1195x; 1.1195x over previous
import jax
import jax.numpy as jnp
from jax.experimental import pallas as pl
from jax.experimental.pallas import tpu as pltpu

_LANE = 128


def _round_up(n, m):
    return ((n + m - 1) // m) * m


def _mlp_kernel(x_ref, w1_ref, b1_ref, w2_ref, b2_ref, wr_ref, br_ref, o_ref):
    h = jnp.dot(x_ref[...], w1_ref[...], preferred_element_type=jnp.float32)
    h = jnp.maximum(h + b1_ref[...], 0.0)
    h = jnp.dot(h, w2_ref[...], preferred_element_type=jnp.float32)
    h = jnp.maximum(h + b2_ref[...], 0.0)
    y = jnp.dot(h, wr_ref[...], preferred_element_type=jnp.float32)
    o_ref[...] = (y + br_ref[...]).astype(o_ref.dtype)


def kernel(x, w1, b1, w2, b2, wr, br):
    B, z_dim = x.shape
    h0_dim, h1_dim, x_dim = w1.shape[1], w2.shape[1], wr.shape[1]

    z_p = _round_up(z_dim, _LANE)
    h0_p = _round_up(h0_dim, _LANE)
    h1_p = _round_up(h1_dim, _LANE)
    x_p = _round_up(x_dim, _LANE)

    tm = 512 if B >= 512 else _round_up(max(B, 1), 8)
    B_p = _round_up(B, tm)
    nb = B_p // tm

    def pad2(a, rows, cols):
        if a.shape == (rows, cols):
            return a
        return jnp.pad(a, ((0, rows - a.shape[0]), (0, cols - a.shape[1])))

    x_pad = pad2(x, B_p, z_p)
    w1_p = pad2(w1, z_p, h0_p)
    w2_p = pad2(w2, h0_p, h1_p)
    wr_p = pad2(wr, h1_p, x_p)
    b1_p = pad2(b1, 1, h0_p)
    b2_p = pad2(b2, 1, h1_p)
    br_p = pad2(br, 1, x_p)

    resident = lambda shape: pl.BlockSpec(shape, lambda i: (0, 0))

    out = pl.pallas_call(
        _mlp_kernel,
        out_shape=jax.ShapeDtypeStruct((B_p, x_p), x.dtype),
        grid=(nb,),
        in_specs=[
            pl.BlockSpec((tm, z_p), lambda i: (i, 0)),
            resident((z_p, h0_p)), resident((1, h0_p)),
            resident((h0_p, h1_p)), resident((1, h1_p)),
            resident((h1_p, x_p)), resident((1, x_p)),
        ],
        out_specs=pl.BlockSpec((tm, x_p), lambda i: (i, 0)),
        compiler_params=pltpu.CompilerParams(
            dimension_semantics=("parallel",),
            vmem_limit_bytes=64 * 1024 * 1024,
        ),
    )(x_pad, w1_p, b1_p, w2_p, b2_p, wr_p, br_p)

    return out[:B, :x_dim]


# speedup vs baseline: 1.1326x; 1.0117x over previous
import jax
import jax.numpy as jnp
from jax.experimental import pallas as pl
from jax.experimental.pallas import tpu as pltpu

_LANE = 128


def _round_up(n, m):
    return ((n + m - 1) // m) * m


def _mlp_kernel(x_ref, w1_ref, b1_ref, w2_ref, b2_ref, wr_ref, br_ref, o_ref):
    h = jnp.dot(x_ref[...], w1_ref[...], preferred_element_type=jnp.float32)
    h = jnp.maximum(h + b1_ref[...], 0.0)
    h = jnp.dot(h, w2_ref[...], preferred_element_type=jnp.float32)
    h = jnp.maximum(h + b2_ref[...], 0.0)
    y = jnp.dot(h, wr_ref[...], preferred_element_type=jnp.float32)
    o_ref[...] = (y + br_ref[...]).astype(o_ref.dtype)


def kernel(x, w1, b1, w2, b2, wr, br):
    B, z_dim = x.shape
    h0_dim, h1_dim, x_dim = w1.shape[1], w2.shape[1], wr.shape[1]

    z_p = _round_up(z_dim, _LANE)
    h0_p = _round_up(h0_dim, _LANE)
    h1_p = _round_up(h1_dim, _LANE)
    x_p = _round_up(x_dim, _LANE)

    tm = 1024 if B >= 1024 else _round_up(max(B, 1), 8)
    B_p = _round_up(B, tm)
    nb = B_p // tm

    def pad2(a, rows, cols):
        if a.shape == (rows, cols):
            return a
        return jnp.pad(a, ((0, rows - a.shape[0]), (0, cols - a.shape[1])))

    x_pad = pad2(x, B_p, z_p)
    w1_p = pad2(w1, z_p, h0_p)
    w2_p = pad2(w2, h0_p, h1_p)
    wr_p = pad2(wr, h1_p, x_p)
    b1_p = pad2(b1, 1, h0_p)
    b2_p = pad2(b2, 1, h1_p)
    br_p = pad2(br, 1, x_p)

    resident = lambda shape: pl.BlockSpec(shape, lambda i: (0, 0))

    out = pl.pallas_call(
        _mlp_kernel,
        out_shape=jax.ShapeDtypeStruct((B_p, x_p), x.dtype),
        grid=(nb,),
        in_specs=[
            pl.BlockSpec((tm, z_p), lambda i: (i, 0)),
            resident((z_p, h0_p)), resident((1, h0_p)),
            resident((h0_p, h1_p)), resident((1, h1_p)),
            resident((h1_p, x_p)), resident((1, x_p)),
        ],
        out_specs=pl.BlockSpec((tm, x_p), lambda i: (i, 0)),
        compiler_params=pltpu.CompilerParams(
            dimension_semantics=("parallel",),
            vmem_limit_bytes=64 * 1024 * 1024,
        ),
    )(x_pad, w1_p, b1_p, w2_p, b2_p, wr_p, br_p)

    return out[:B, :x_dim]
